# baseline (speedup 1.0000x reference)
import functools

import jax
import jax.numpy as jnp
from jax import lax
from jax.experimental import pallas as pl
from jax.experimental.pallas import tpu as pltpu
from jax.experimental.pallas import tpu_sc as plsc

TN = 2048
NW_USED = 8
GRP = 16


@functools.lru_cache(maxsize=None)
def _make_sc_gather(Vrows, D, B):
    per_w = B // NW_USED
    n_grp = per_w // GRP
    mesh = plsc.VectorSubcoreMesh(core_axis_name="c", subcore_axis_name="s")

    @functools.partial(
        pl.kernel,
        mesh=mesh,
        out_type=jax.ShapeDtypeStruct((D, B), jnp.float32),
        scratch_types=[
            pltpu.VMEM((per_w,), jnp.int32),
            pltpu.VMEM((GRP, D, 128), jnp.float32),
            pltpu.VMEM((D, per_w), jnp.float32),
            pltpu.SemaphoreType.DMA,
        ],
        compiler_params=pltpu.CompilerParams(use_tc_tiling_on_sc=True, needs_layout_passes=False),
    )
    def gather(embT_hbm, idx_hbm, out_hbm, idx_v, blks, cols, sem):
        wid = lax.axis_index("s") * 2 + lax.axis_index("c")

        @pl.when(wid < NW_USED)
        def _():
            base_i = wid * per_w
            pltpu.sync_copy(idx_hbm.at[pl.ds(base_i, per_w)], idx_v)
            lanes = lax.iota(jnp.int32, 16)
            for g in range(n_grp):
                iv = idx_v[pl.ds(g * GRP, GRP)]
                us = []
                descs = []
                for l in range(GRP):
                    u = jnp.max(jnp.where(lanes == l, iv, 0))
                    us.append(u)
                    blk_base = (u // 128) * 128
                    descs.append(pltpu.async_copy(
                        embT_hbm.at[:, pl.ds(blk_base, 128)],
                        blks.at[l], sem))
                for d in descs:
                    d.wait()
                for l in range(GRP):
                    cvec = jnp.full((16,), lax.rem(us[l], 128), jnp.int32)
                    col = plsc.load_gather(blks.at[l], [lanes, cvec])
                    pvec = jnp.full((16,), g * GRP + l, jnp.int32)
                    plsc.store_scatter(cols, [lanes, pvec], col)
            pltpu.sync_copy(cols, out_hbm.at[:, pl.ds(base_i, per_w)])

    return gather


def _mm_body(uet_ref, wut_ref, wht_ref, dect_ref, out_ref):
    dn = (((0,), (0,)), ((), ()))
    u = lax.dot_general(wut_ref[...], uet_ref[...], dn,
                        preferred_element_type=jnp.float32)
    h = lax.dot_general(wht_ref[...], dect_ref[...], dn,
                        preferred_element_type=jnp.float32)
    out_ref[...] = u + h


@functools.lru_cache(maxsize=None)
def _make_tc_matmul(B, D, V):
    grid = (pl.cdiv(V, TN),)
    return pl.pallas_call(
        _mm_body,
        grid=grid,
        in_specs=[
            pl.BlockSpec((D, B), lambda j: (0, 0)),
            pl.BlockSpec((D, TN), lambda j: (0, j)),
            pl.BlockSpec((D, TN), lambda j: (0, j)),
            pl.BlockSpec((D, B), lambda j: (0, 0)),
        ],
        out_specs=pl.BlockSpec((TN, B), lambda j: (j, 0)),
        out_shape=jax.ShapeDtypeStruct((V, B), jnp.float32),
        compiler_params=pltpu.CompilerParams(
            dimension_semantics=("arbitrary",),
        ),
    )


def kernel(user, decoder_output, embedding, W_u, W_h):
    V, D = W_u.shape
    B = user.shape[0]
    embT = embedding.T
    wut = W_u.T
    wht = W_h.T
    dect = jnp.squeeze(decoder_output, axis=0).T
    uet = _make_sc_gather(embedding.shape[0], D, B)(
        embT, user.astype(jnp.int32))
    out_t = _make_tc_matmul(B, D, V)(uet, wut, wht, dect)
    return out_t.T


# speedup vs baseline: 1.0248x; 1.0248x over previous
import functools

import jax
import jax.numpy as jnp
from jax import lax
from jax.experimental import pallas as pl
from jax.experimental.pallas import tpu as pltpu
from jax.experimental.pallas import tpu_sc as plsc

TN = 4096
NW_USED = 8
GRP = 16


@functools.lru_cache(maxsize=None)
def _make_sc_gather(Vrows, D, B):
    per_w = B // NW_USED
    n_grp = per_w // GRP
    mesh = plsc.VectorSubcoreMesh(core_axis_name="c", subcore_axis_name="s")

    @functools.partial(
        pl.kernel,
        mesh=mesh,
        out_type=jax.ShapeDtypeStruct((D, B), jnp.float32),
        scratch_types=[
            pltpu.VMEM((per_w,), jnp.int32),
            pltpu.VMEM((GRP, D, 128), jnp.float32),
            pltpu.VMEM((D, per_w), jnp.float32),
            pltpu.SemaphoreType.DMA,
        ],
        compiler_params=pltpu.CompilerParams(use_tc_tiling_on_sc=True, needs_layout_passes=False),
    )
    def gather(embT_hbm, idx_hbm, out_hbm, idx_v, blks, cols, sem):
        wid = lax.axis_index("s") * 2 + lax.axis_index("c")

        @pl.when(wid < NW_USED)
        def _():
            base_i = wid * per_w
            pltpu.sync_copy(idx_hbm.at[pl.ds(base_i, per_w)], idx_v)
            lanes = lax.iota(jnp.int32, 16)
            for g in range(n_grp):
                iv = idx_v[pl.ds(g * GRP, GRP)]
                us = []
                descs = []
                for l in range(GRP):
                    u = jnp.max(jnp.where(lanes == l, iv, 0))
                    us.append(u)
                    blk_base = (u // 128) * 128
                    descs.append(pltpu.async_copy(
                        embT_hbm.at[:, pl.ds(blk_base, 128)],
                        blks.at[l], sem))
                for d in descs:
                    d.wait()
                for l in range(GRP):
                    cvec = jnp.full((16,), lax.rem(us[l], 128), jnp.int32)
                    col = plsc.load_gather(blks.at[l], [lanes, cvec])
                    pvec = jnp.full((16,), g * GRP + l, jnp.int32)
                    plsc.store_scatter(cols, [lanes, pvec], col)
            pltpu.sync_copy(cols, out_hbm.at[:, pl.ds(base_i, per_w)])

    return gather


def _mm_body(uet_ref, wut_ref, wht_ref, dect_ref, out_ref):
    dn = (((0,), (0,)), ((), ()))
    u = lax.dot_general(wut_ref[...], uet_ref[...], dn,
                        preferred_element_type=jnp.float32)
    h = lax.dot_general(wht_ref[...], dect_ref[...], dn,
                        preferred_element_type=jnp.float32)
    out_ref[...] = u + h


@functools.lru_cache(maxsize=None)
def _make_tc_matmul(B, D, V):
    grid = (pl.cdiv(V, TN),)
    return pl.pallas_call(
        _mm_body,
        grid=grid,
        in_specs=[
            pl.BlockSpec((D, B), lambda j: (0, 0)),
            pl.BlockSpec((D, TN), lambda j: (0, j)),
            pl.BlockSpec((D, TN), lambda j: (0, j)),
            pl.BlockSpec((D, B), lambda j: (0, 0)),
        ],
        out_specs=pl.BlockSpec((TN, B), lambda j: (j, 0)),
        out_shape=jax.ShapeDtypeStruct((V, B), jnp.float32),
        compiler_params=pltpu.CompilerParams(
            dimension_semantics=("arbitrary",),
        ),
    )


def kernel(user, decoder_output, embedding, W_u, W_h):
    V, D = W_u.shape
    B = user.shape[0]
    embT = embedding.T
    wut = W_u.T
    wht = W_h.T
    dect = jnp.squeeze(decoder_output, axis=0).T
    uet = _make_sc_gather(embedding.shape[0], D, B)(
        embT, user.astype(jnp.int32))
    out_t = _make_tc_matmul(B, D, V)(uet, wut, wht, dect)
    return out_t.T


# speedup vs baseline: 1.1182x; 1.0912x over previous
import functools

import jax
import jax.numpy as jnp
from jax import lax
from jax.experimental import pallas as pl
from jax.experimental.pallas import tpu as pltpu
from jax.experimental.pallas import tpu_sc as plsc

TN = 4096
GRP = 16


@functools.lru_cache(maxsize=None)
def _make_sc_gather(Vrows, D, B):
    NW = 32
    per_w = B // NW
    n_grp = per_w // GRP
    mesh = plsc.VectorSubcoreMesh(core_axis_name="c", subcore_axis_name="s")

    @functools.partial(
        pl.kernel,
        mesh=mesh,
        out_type=jax.ShapeDtypeStruct((NW, D, per_w), jnp.float32),
        scratch_types=[
            pltpu.VMEM((B,), jnp.int32),
            pltpu.VMEM((GRP, D, 128), jnp.float32),
            pltpu.VMEM((D, per_w), jnp.float32),
            pltpu.SemaphoreType.DMA,
        ],
        compiler_params=pltpu.CompilerParams(use_tc_tiling_on_sc=True, needs_layout_passes=False),
    )
    def gather(embT_hbm, idx_hbm, out_hbm, idx_v, blks, cols, sem):
        wid = lax.axis_index("s") * 2 + lax.axis_index("c")
        base_i = wid * per_w
        pltpu.sync_copy(idx_hbm, idx_v)
        lanes = lax.iota(jnp.int32, 16)
        for g in range(n_grp):
            iv = idx_v[pl.ds(base_i + g * GRP, GRP)]
            us = []
            descs = []
            for l in range(GRP):
                u = jnp.max(jnp.where(lanes == l, iv, 0))
                us.append(u)
                blk_base = (u // 128) * 128
                descs.append(pltpu.async_copy(
                    embT_hbm.at[:, pl.ds(blk_base, 128)],
                    blks.at[l], sem))
            for d in descs:
                d.wait()
            for l in range(GRP):
                cvec = jnp.full((16,), lax.rem(us[l], 128), jnp.int32)
                col = plsc.load_gather(blks.at[l], [lanes, cvec])
                pvec = jnp.full((16,), g * GRP + l, jnp.int32)
                plsc.store_scatter(cols, [lanes, pvec], col)
        pltpu.sync_copy(cols, out_hbm.at[wid])

    return gather


def _mm_body(uet_ref, wut_ref, wht_ref, dect_ref, out_ref):
    dn = (((0,), (0,)), ((), ()))
    u = lax.dot_general(wut_ref[...], uet_ref[...], dn,
                        preferred_element_type=jnp.float32)
    h = lax.dot_general(wht_ref[...], dect_ref[...], dn,
                        preferred_element_type=jnp.float32)
    out_ref[...] = u + h


@functools.lru_cache(maxsize=None)
def _make_tc_matmul(B, D, V):
    grid = (pl.cdiv(V, TN),)
    return pl.pallas_call(
        _mm_body,
        grid=grid,
        in_specs=[
            pl.BlockSpec((D, B), lambda j: (0, 0)),
            pl.BlockSpec((D, TN), lambda j: (0, j)),
            pl.BlockSpec((D, TN), lambda j: (0, j)),
            pl.BlockSpec((D, B), lambda j: (0, 0)),
        ],
        out_specs=pl.BlockSpec((TN, B), lambda j: (j, 0)),
        out_shape=jax.ShapeDtypeStruct((V, B), jnp.float32),
        compiler_params=pltpu.CompilerParams(
            dimension_semantics=("arbitrary",),
        ),
    )


def kernel(user, decoder_output, embedding, W_u, W_h):
    V, D = W_u.shape
    B = user.shape[0]
    embT = embedding.T
    wut = W_u.T
    wht = W_h.T
    dect = jnp.squeeze(decoder_output, axis=0).T
    uet3 = _make_sc_gather(embedding.shape[0], D, B)(
        embT, user.astype(jnp.int32))
    uet = uet3.transpose(1, 0, 2).reshape(D, B)
    out_t = _make_tc_matmul(B, D, V)(uet, wut, wht, dect)
    return out_t.T


# speedup vs baseline: 1.1215x; 1.0029x over previous
import functools

import jax
import jax.numpy as jnp
from jax import lax
from jax.experimental import pallas as pl
from jax.experimental.pallas import tpu as pltpu
from jax.experimental.pallas import tpu_sc as plsc

TN = 4096
GRP = 16


@functools.lru_cache(maxsize=None)
def _make_sc_gather(Vrows, D, B):
    NW = 32
    per_w = B // NW
    n_grp = per_w // GRP
    mesh = plsc.VectorSubcoreMesh(core_axis_name="c", subcore_axis_name="s")

    @functools.partial(
        pl.kernel,
        mesh=mesh,
        out_type=jax.ShapeDtypeStruct((NW, D, per_w), jnp.float32),
        scratch_types=[
            pltpu.VMEM((B,), jnp.int32),
            pltpu.VMEM((GRP, D, 128), jnp.float32),
            pltpu.VMEM((D, per_w), jnp.float32),
            pltpu.SemaphoreType.DMA,
        ],
        compiler_params=pltpu.CompilerParams(use_tc_tiling_on_sc=True, needs_layout_passes=False),
    )
    def gather(embT_hbm, idx_hbm, out_hbm, idx_v, blks, cols, sem):
        wid = lax.axis_index("s") * 2 + lax.axis_index("c")
        base_i = wid * per_w
        pltpu.sync_copy(idx_hbm, idx_v)
        lanes = lax.iota(jnp.int32, 16)
        for g in range(n_grp):
            iv = idx_v[pl.ds(base_i + g * GRP, GRP)]
            us = []
            descs = []
            for l in range(GRP):
                u = jnp.max(jnp.where(lanes == l, iv, 0))
                us.append(u)
                blk_base = (u // 128) * 128
                descs.append(pltpu.async_copy(
                    embT_hbm.at[:, pl.ds(blk_base, 128)],
                    blks.at[l], sem))
            for d in descs:
                d.wait()
            for l in range(GRP):
                cvec = jnp.full((16,), lax.rem(us[l], 128), jnp.int32)
                col = plsc.load_gather(blks.at[l], [lanes, cvec])
                pvec = jnp.full((16,), g * GRP + l, jnp.int32)
                plsc.store_scatter(cols, [lanes, pvec], col)
        pltpu.sync_copy(cols, out_hbm.at[wid])

    return gather


def _mm_body(uet_ref, wut_ref, wht_ref, dect_ref, out_ref):
    dn = (((0,), (0,)), ((), ()))
    u = lax.dot_general(wut_ref[...], uet_ref[...], dn,
                        preferred_element_type=jnp.float32)
    h = lax.dot_general(wht_ref[...], dect_ref[...], dn,
                        preferred_element_type=jnp.float32)
    out_ref[...] = u + h


@functools.lru_cache(maxsize=None)
def _make_tc_matmul(B, D, V):
    grid = (pl.cdiv(V, TN),)
    return pl.pallas_call(
        _mm_body,
        grid=grid,
        in_specs=[
            pl.BlockSpec((D, B), lambda j: (0, 0)),
            pl.BlockSpec((D, TN), lambda j: (0, j)),
            pl.BlockSpec((D, TN), lambda j: (0, j)),
            pl.BlockSpec((D, B), lambda j: (0, 0)),
        ],
        out_specs=pl.BlockSpec((TN, B), lambda j: (j, 0)),
        out_shape=jax.ShapeDtypeStruct((V, B), jnp.float32),
        compiler_params=pltpu.CompilerParams(
            dimension_semantics=("arbitrary",),
        ),
    )


def kernel(user, decoder_output, embedding, W_u, W_h):
    V, D = W_u.shape
    B = user.shape[0]
    embT = embedding.T
    wut = W_u.T.astype(jnp.bfloat16)
    wht = W_h.T.astype(jnp.bfloat16)
    dect = jnp.squeeze(decoder_output, axis=0).T.astype(jnp.bfloat16)
    uet3 = _make_sc_gather(embedding.shape[0], D, B)(
        embT, user.astype(jnp.int32))
    uet = uet3.transpose(1, 0, 2).reshape(D, B).astype(jnp.bfloat16)
    out_t = _make_tc_matmul(B, D, V)(uet, wut, wht, dect)
    return out_t.T
